# baseline (speedup 1.0000x reference)
import jax
import jax.numpy as jnp
from jax import lax
from jax.experimental import pallas as pl
from jax.experimental.pallas import tpu as pltpu
from jax.experimental.pallas import tpu_sc as plsc

N = 16384
C = 512
R = 4
C2 = C // R

NC = 2
NS = 16
NW = NC * NS
LANES = 16

ROWS_PER_W = N // NW
CIN = 64
COUT = 32
N_IN = ROWS_PER_W // CIN
VREGS_PER_ROW = C // LANES


def _full(val):
    return jnp.full((LANES,), val, dtype=jnp.int32)


def _body(x_hbm, out_hbm, in0, in1, ot0, ot1, si0, si1, so0, so1):
    wid = lax.axis_index("s") * NC + lax.axis_index("c")
    row0 = wid * ROWS_PER_W

    lane = lax.iota(jnp.int32, LANES)
    lane_mod = lax.rem(lane, _full(R))
    col_k = [lax.div(lane, _full(R)) + _full(4 * k)
             for k in range(VREGS_PER_ROW)]

    def in_copy(g, buf, sem):
        return pltpu.async_copy(
            x_hbm.at[pl.ds(row0 + g * CIN, CIN), :], buf, sem)

    def out_copy(g, h, buf, sem):
        return pltpu.async_copy(
            buf,
            out_hbm.at[pl.ds((row0 + g * CIN + h * COUT) * R, COUT * R), :],
            sem)

    def permute(in_v, h, out_v):
        @plsc.parallel_loop(0, COUT, unroll=4)
        def row_body(n):
            rvec = jnp.full((LANES,), R * n, dtype=jnp.int32) + lane_mod
            for k in range(VREGS_PER_ROW):
                v = in_v[h * COUT + n, pl.ds(16 * k, LANES)]
                plsc.store_scatter(out_v, [rvec, col_k[k]], v)

    in_bufs = (in0, in1)
    in_sems = (si0, si1)
    out_bufs = (ot0, ot1)
    out_sems = (so0, so1)

    in_copy(0, in0, si0)
    in_copy(1, in1, si1)

    def pair_body(i, carry):
        g = 2 * i

        def stage(g, in_v, si, odd):
            pltpu.make_async_copy(
                x_hbm.at[pl.ds(0, CIN), :], in_v, si).wait()
            for h in range(2):
                out_v, so = out_bufs[h], out_sems[h]

                def drain():
                    pltpu.make_async_copy(
                        out_v, out_hbm.at[pl.ds(0, COUT * R), :], so).wait()

                if odd:
                    drain()
                else:
                    pl.when(i > 0)(drain)
                permute(in_v, h, out_v)
                out_copy(g, h, out_v, so)
            @pl.when(i < N_IN // 2 - 1)
            def _():
                in_copy(g + 2, in_v, si)

        stage(g, in0, si0, False)
        stage(g + 1, in1, si1, True)
        return carry

    lax.fori_loop(0, N_IN // 2, pair_body, 0)

    pltpu.make_async_copy(ot0, out_hbm.at[pl.ds(0, COUT * R), :], so0).wait()
    pltpu.make_async_copy(ot1, out_hbm.at[pl.ds(0, COUT * R), :], so1).wait()


@jax.jit
def _point_shuffle(x):
    mesh = plsc.VectorSubcoreMesh(core_axis_name="c", subcore_axis_name="s")
    run = pl.kernel(
        _body,
        out_type=jax.ShapeDtypeStruct((N * R, C2), jnp.float32),
        mesh=mesh,
        scratch_types=[
            pltpu.VMEM((CIN, C), jnp.float32),
            pltpu.VMEM((CIN, C), jnp.float32),
            pltpu.VMEM((COUT * R, C2), jnp.float32),
            pltpu.VMEM((COUT * R, C2), jnp.float32),
            pltpu.SemaphoreType.DMA,
            pltpu.SemaphoreType.DMA,
            pltpu.SemaphoreType.DMA,
            pltpu.SemaphoreType.DMA,
        ],
        compiler_params=pltpu.CompilerParams(needs_layout_passes=False),
    )
    return run(x)


def kernel(x):
    return _point_shuffle(x)


# speedup vs baseline: 1.0433x; 1.0433x over previous
import jax
import jax.numpy as jnp
from jax import lax
from jax.experimental import pallas as pl
from jax.experimental.pallas import tpu as pltpu
from jax.experimental.pallas import tpu_sc as plsc

N = 16384
C = 512
R = 4
C2 = C // R

NC = 2
NS = 16
NW = NC * NS
LANES = 16

ROWS_PER_W = N // NW
CHUNK = 32
N_CHUNKS = ROWS_PER_W // CHUNK
N_QUADS = N_CHUNKS // 4
VREGS_PER_ROW = C // LANES


def _full(val):
    return jnp.full((LANES,), val, dtype=jnp.int32)


def _body(x_hbm, out_hbm, in0, in1, in2, in3, ot0, ot1,
          si0, si1, si2, si3, so0, so1):
    wid = lax.axis_index("s") * NC + lax.axis_index("c")
    row0 = wid * ROWS_PER_W

    lane = lax.iota(jnp.int32, LANES)
    lane_mod = lax.rem(lane, _full(R))
    col_k = [lax.div(lane, _full(R)) + _full(4 * k)
             for k in range(VREGS_PER_ROW)]

    def in_copy(g, buf, sem):
        return pltpu.async_copy(
            x_hbm.at[pl.ds(row0 + g * CHUNK, CHUNK), :], buf, sem)

    def out_copy(g, buf, sem):
        return pltpu.async_copy(
            buf, out_hbm.at[pl.ds((row0 + g * CHUNK) * R, CHUNK * R), :],
            sem)

    def permute(in_v, out_v):
        @plsc.parallel_loop(0, CHUNK, unroll=2)
        def row_body(n):
            rvec = jnp.full((LANES,), R * n, dtype=jnp.int32) + lane_mod
            for k in range(VREGS_PER_ROW):
                v = in_v[n, pl.ds(16 * k, LANES)]
                plsc.store_scatter(out_v, [rvec, col_k[k]], v)

    in_bufs = (in0, in1, in2, in3)
    in_sems = (si0, si1, si2, si3)
    out_bufs = (ot0, ot1)
    out_sems = (so0, so1)

    for j in range(4):
        in_copy(j, in_bufs[j], in_sems[j])

    def quad_body(i, carry):
        g = 4 * i

        def stage(q, gq):
            in_v, si = in_bufs[q], in_sems[q]
            out_v, so = out_bufs[q % 2], out_sems[q % 2]
            pltpu.make_async_copy(
                x_hbm.at[pl.ds(0, CHUNK), :], in_v, si).wait()
            if q >= 2:
                pltpu.make_async_copy(
                    out_v, out_hbm.at[pl.ds(0, CHUNK * R), :], so).wait()
            else:
                @pl.when(i > 0)
                def _():
                    pltpu.make_async_copy(
                        out_v, out_hbm.at[pl.ds(0, CHUNK * R), :], so).wait()
            permute(in_v, out_v)
            out_copy(gq, out_v, so)
            @pl.when(i < N_QUADS - 1)
            def _():
                in_copy(gq + 4, in_v, si)

        for q in range(4):
            stage(q, g + q)
        return carry

    lax.fori_loop(0, N_QUADS, quad_body, 0)

    pltpu.make_async_copy(ot0, out_hbm.at[pl.ds(0, CHUNK * R), :], so0).wait()
    pltpu.make_async_copy(ot1, out_hbm.at[pl.ds(0, CHUNK * R), :], so1).wait()


@jax.jit
def _point_shuffle(x):
    mesh = plsc.VectorSubcoreMesh(core_axis_name="c", subcore_axis_name="s")
    run = pl.kernel(
        _body,
        out_type=jax.ShapeDtypeStruct((N * R, C2), jnp.float32),
        mesh=mesh,
        scratch_types=[
            pltpu.VMEM((CHUNK, C), jnp.float32),
            pltpu.VMEM((CHUNK, C), jnp.float32),
            pltpu.VMEM((CHUNK, C), jnp.float32),
            pltpu.VMEM((CHUNK, C), jnp.float32),
            pltpu.VMEM((CHUNK * R, C2), jnp.float32),
            pltpu.VMEM((CHUNK * R, C2), jnp.float32),
            pltpu.SemaphoreType.DMA,
            pltpu.SemaphoreType.DMA,
            pltpu.SemaphoreType.DMA,
            pltpu.SemaphoreType.DMA,
            pltpu.SemaphoreType.DMA,
            pltpu.SemaphoreType.DMA,
        ],
        compiler_params=pltpu.CompilerParams(needs_layout_passes=False),
    )
    return run(x)


def kernel(x):
    return _point_shuffle(x)
